# Initial kernel scaffold; baseline (speedup 1.0000x reference)
import jax
import jax.numpy as jnp
from jax.experimental import pallas as pl


def kernel(x, pos, batch, W_lin, W_src, W_dst, W_p1, b_p1, W_p2, b_p2, W_attn, b_attn):
    raise NotImplementedError("write your pallas kernel here")


# speedup vs baseline: 4.3228x; 4.3228x over previous
import functools

import jax
import jax.numpy as jnp
from jax.experimental import pallas as pl

MAXNB = 64
R2 = 2.5 * 2.5
BIG = 1e30
NEG = -1e30


def _fps_body(pos_ref, q_ref, *, n, m):
    pos = pos_ref[:]
    lane = jax.lax.broadcasted_iota(jnp.int32, (1, n), 1)

    def extract(j):
        return jnp.sum(jnp.where(lane == j, pos, 0.0), axis=1, keepdims=True)

    qlane = jax.lax.broadcasted_iota(jnp.int32, (1, m), 1)
    q0 = jnp.where(qlane == 0, extract(jnp.int32(0)), jnp.zeros((6, m), jnp.float32))

    def body(i, carry):
        mind, qacc, last = carry
        sel = extract(last)
        d = jnp.sum((pos - sel) ** 2, axis=0, keepdims=True)
        mind = jnp.minimum(mind, d)
        mval = jnp.max(mind)
        nxt = jnp.min(jnp.where(mind == mval, lane, n)).astype(jnp.int32)
        qacc = jnp.where(qlane == i, extract(nxt), qacc)
        return mind, qacc, nxt

    mind0 = jnp.full((1, n), BIG, jnp.float32)
    _, qacc, _ = jax.lax.fori_loop(1, m, body, (mind0, q0, jnp.int32(0)))
    q_ref[:] = qacc


def _radius_body(q_ref, pos_ref, cols_ref, mask_ref, *, n, bq):
    d2 = jnp.zeros((n, bq), jnp.float32)
    for d in range(6):
        pd = pos_ref[:, d : d + 1] - q_ref[d : d + 1, :]
        d2 = d2 + pd * pd
    d2 = jnp.where(d2 <= R2, d2, BIG)
    sub = jax.lax.broadcasted_iota(jnp.int32, (n, 1), 0)
    krow = jax.lax.broadcasted_iota(jnp.int32, (MAXNB, 1), 0)
    rowid = pl.program_id(0) * bq + jax.lax.broadcasted_iota(jnp.int32, (1, bq), 1)

    def body(k, carry):
        d2c, cols, msk = carry
        m = jnp.min(d2c, axis=0, keepdims=True)
        idx = jnp.min(jnp.where(d2c == m, sub, n), axis=0, keepdims=True).astype(
            jnp.int32
        )
        valid = jnp.where((m < BIG * 0.5) & (idx != rowid), 1.0, 0.0)
        cols = jnp.where(krow == k, idx, cols)
        msk = jnp.where(krow == k, valid, msk)
        d2c = jnp.where(sub == idx, BIG, d2c)
        return d2c, cols, msk

    cols0 = jnp.zeros((MAXNB, bq), jnp.int32)
    msk0 = jnp.zeros((MAXNB, bq), jnp.float32)
    _, cols, msk = jax.lax.fori_loop(0, MAXNB, body, (d2, cols0, msk0))
    cols_ref[:] = cols
    mask_ref[:] = msk


def _dense_body(x_ref, wl_ref, ws_ref, wd_ref, bp1_ref, wp2_ref, bp2_ref, wa_ref,
                ba_ref, xv_ref, as_ref, ad_ref, al_ref, ml_ref):
    x = x_ref[:]
    f32 = jnp.float32
    xv = jnp.dot(x, wl_ref[:], preferred_element_type=f32)
    a_s = jnp.dot(x, ws_ref[:], preferred_element_type=f32)
    a_d = jnp.dot(x, wd_ref[:], preferred_element_type=f32)
    d0 = jax.nn.relu(
        jnp.dot(jax.nn.relu(bp1_ref[:]), wp2_ref[:], preferred_element_type=f32)
        + bp2_ref[:]
    )
    al = jnp.dot(a_d - a_s + d0, wa_ref[:], preferred_element_type=f32) + ba_ref[:]
    xv_ref[:] = xv
    as_ref[:] = a_s
    ad_ref[:] = a_d
    al_ref[:] = al
    ml_ref[:] = xv + d0


def _edges_body(pr_ref, pc_ref, adr_ref, asc_ref, xvc_ref, mask_ref, al_ref,
                ml_ref, wp1_ref, bp1_ref, wp2_ref, bp2_ref, wa_ref, ba_ref,
                out_ref, *, q):
    f32 = jnp.float32
    pd = pr_ref[:] - pc_ref[:]
    h = jax.nn.relu(jnp.dot(pd, wp1_ref[:], preferred_element_type=f32) + bp1_ref[:])
    delta = jax.nn.relu(
        jnp.dot(h, wp2_ref[:], preferred_element_type=f32) + bp2_ref[:]
    )
    alpha = (
        jnp.dot(adr_ref[:] - asc_ref[:] + delta, wa_ref[:], preferred_element_type=f32)
        + ba_ref[:]
    )
    msk = mask_ref[:]
    alpha = jnp.where(msk > 0.5, alpha, NEG)
    a3 = alpha.reshape(q, MAXNB, 128)
    m3 = msk.reshape(q, MAXNB, 1)
    al = al_ref[:]
    amax = jnp.maximum(jnp.max(a3, axis=1), al)
    ex3 = jnp.exp(a3 - amax[:, None, :]) * m3
    exl = jnp.exp(al - amax)
    den = jnp.sum(ex3, axis=1) + exl
    xv3 = xvc_ref[:].reshape(q, MAXNB, 128)
    d3 = delta.reshape(q, MAXNB, 128)
    num = jnp.sum(ex3 * (xv3 + d3), axis=1) + exl * ml_ref[:]
    out_ref[:] = num / jnp.maximum(den, 1e-16)


def kernel(x, pos, batch, W_lin, W_src, W_dst, W_p1, b_p1, W_p2, b_p2, W_attn, b_attn):
    n, d_in = x.shape
    m = n // 4
    f32 = jnp.float32
    pos_t = pos.T

    q_t = pl.pallas_call(
        functools.partial(_fps_body, n=n, m=m),
        out_shape=jax.ShapeDtypeStruct((6, m), f32),
    )(pos_t)

    mp = 2560
    q_p = jnp.pad(q_t, ((0, 0), (0, mp - m)), constant_values=1e9)
    bq = 128
    cols_t, mask_t = pl.pallas_call(
        functools.partial(_radius_body, n=n, bq=bq),
        grid=(mp // bq,),
        in_specs=[
            pl.BlockSpec((6, bq), lambda i: (0, i)),
            pl.BlockSpec((n, 6), lambda i: (0, 0)),
        ],
        out_specs=[
            pl.BlockSpec((MAXNB, bq), lambda i: (0, i)),
            pl.BlockSpec((MAXNB, bq), lambda i: (0, i)),
        ],
        out_shape=[
            jax.ShapeDtypeStruct((MAXNB, mp), jnp.int32),
            jax.ShapeDtypeStruct((MAXNB, mp), f32),
        ],
    )(q_p, pos)

    bp1 = b_p1.reshape(1, -1)
    bp2 = b_p2.reshape(1, -1)
    ba = b_attn.reshape(1, -1)
    bn = 1000
    full = lambda a: pl.BlockSpec(a.shape, lambda i: (0, 0))
    xv, a_s, a_d, al, ml = pl.pallas_call(
        _dense_body,
        grid=(n // bn,),
        in_specs=[
            pl.BlockSpec((bn, d_in), lambda i: (i, 0)),
            full(W_lin), full(W_src), full(W_dst), full(bp1), full(W_p2),
            full(bp2), full(W_attn), full(ba),
        ],
        out_specs=[pl.BlockSpec((bn, 128), lambda i: (i, 0))] * 5,
        out_shape=[jax.ShapeDtypeStruct((n, 128), f32)] * 5,
    )(x, W_lin, W_src, W_dst, bp1, W_p2, bp2, W_attn, ba)

    cols = cols_t.T.reshape(-1)
    maskf = mask_t.T.reshape(-1, 1)
    pos_c = jnp.take(pos, cols, axis=0)
    as_c = jnp.take(a_s, cols, axis=0)
    xv_c = jnp.take(xv, cols, axis=0)
    padq = lambda a: jnp.pad(a[:m], ((0, mp - m), (0, 0)))
    pos_r = jnp.repeat(padq(pos), MAXNB, axis=0)
    ad_r = jnp.repeat(padq(a_d), MAXNB, axis=0)
    al_p = padq(al)
    ml_p = padq(ml)

    qb = 32
    e = qb * MAXNB
    out_head = pl.pallas_call(
        functools.partial(_edges_body, q=qb),
        grid=(mp // qb,),
        in_specs=[
            pl.BlockSpec((e, 6), lambda i: (i, 0)),
            pl.BlockSpec((e, 6), lambda i: (i, 0)),
            pl.BlockSpec((e, 128), lambda i: (i, 0)),
            pl.BlockSpec((e, 128), lambda i: (i, 0)),
            pl.BlockSpec((e, 128), lambda i: (i, 0)),
            pl.BlockSpec((e, 1), lambda i: (i, 0)),
            pl.BlockSpec((qb, 128), lambda i: (i, 0)),
            pl.BlockSpec((qb, 128), lambda i: (i, 0)),
            full(W_p1), full(bp1), full(W_p2), full(bp2), full(W_attn), full(ba),
        ],
        out_specs=pl.BlockSpec((qb, 128), lambda i: (i, 0)),
        out_shape=jax.ShapeDtypeStruct((mp, 128), f32),
    )(pos_r, pos_c, ad_r, as_c, xv_c, maskf, al_p, ml_p,
      W_p1, bp1, W_p2, bp2, W_attn, ba)

    out = jnp.concatenate([out_head[:m], ml[m:]], axis=0)
    return (out, pos, batch)
